# Initial kernel scaffold; baseline (speedup 1.0000x reference)
import jax
import jax.numpy as jnp
from jax.experimental import pallas as pl


def kernel(x, table):
    raise NotImplementedError("write your pallas kernel here")


# speedup vs baseline: 2.9628x; 2.9628x over previous
import functools

import jax
import jax.numpy as jnp
from jax import lax
from jax.experimental import pallas as pl
from jax.experimental.pallas import tpu as pltpu
from jax.experimental.pallas import tpu_sc as plsc

D_MODEL = 128
CHUNK = 128


@jax.jit
def kernel(x, table):
    batch, seq = x.shape
    n_idx = batch * seq
    d = table.shape[1]

    info = plsc.get_sparse_core_info()
    n_workers = info.num_cores * info.num_subcores
    per_w = n_idx // n_workers
    n_chunks = per_w // CHUNK
    assert per_w * n_workers == n_idx and n_chunks * CHUNK == per_w

    idx = x.reshape(n_workers, n_chunks, CHUNK).astype(jnp.int32)
    mesh = plsc.VectorSubcoreMesh(core_axis_name="c", subcore_axis_name="s")

    @functools.partial(
        pl.kernel,
        mesh=mesh,
        out_type=jax.ShapeDtypeStruct((n_idx, d), jnp.float32),
        scratch_types=[
            pltpu.VMEM((n_chunks, CHUNK), jnp.int32),
            pltpu.VMEM((CHUNK, D_MODEL), jnp.float32),
            pltpu.SemaphoreType.DMA,
        ],
    )
    def gather_kernel(idx_hbm, table_hbm, out_hbm, idx_v, rows_v, sem):
        wid = lax.axis_index("s") * info.num_cores + lax.axis_index("c")
        base = wid * per_w
        pltpu.sync_copy(idx_hbm.at[wid], idx_v)

        def body(j, _):
            pltpu.async_copy(table_hbm.at[idx_v.at[j]], rows_v, sem).wait()
            pltpu.sync_copy(rows_v, out_hbm.at[pl.ds(base + j * CHUNK, CHUNK)])
            return ()

        lax.fori_loop(0, n_chunks, body, ())

    out = gather_kernel(idx, table)
    return out.reshape(batch, seq, d)


# speedup vs baseline: 3.1219x; 1.0537x over previous
import functools

import jax
import jax.numpy as jnp
from jax import lax
from jax.experimental import pallas as pl
from jax.experimental.pallas import tpu as pltpu
from jax.experimental.pallas import tpu_sc as plsc

D_MODEL = 128
CHUNK = 128


@jax.jit
def kernel(x, table):
    batch, seq = x.shape
    n_idx = batch * seq
    d = table.shape[1]

    info = plsc.get_sparse_core_info()
    n_workers = info.num_cores * info.num_subcores
    per_w = n_idx // n_workers
    n_chunks = per_w // CHUNK
    assert per_w * n_workers == n_idx and n_chunks * CHUNK == per_w

    idx = x.reshape(n_workers, n_chunks, CHUNK).astype(jnp.int32)
    mesh = plsc.VectorSubcoreMesh(core_axis_name="c", subcore_axis_name="s")

    @functools.partial(
        pl.kernel,
        mesh=mesh,
        out_type=jax.ShapeDtypeStruct((n_idx, d), jnp.float32),
        scratch_types=[
            pltpu.VMEM((n_chunks, CHUNK), jnp.int32),
            pltpu.VMEM((2, CHUNK, D_MODEL), jnp.float32),
            pltpu.SemaphoreType.DMA,
            pltpu.SemaphoreType.DMA,
            pltpu.SemaphoreType.DMA,
            pltpu.SemaphoreType.DMA,
        ],
    )
    def gather_kernel(
        idx_hbm, table_hbm, out_hbm, idx_v, rows_v, g0, g1, o0, o1
    ):
        wid = lax.axis_index("s") * info.num_cores + lax.axis_index("c")
        base = wid * per_w
        gsems = (g0, g1)
        osems = (o0, o1)
        pltpu.sync_copy(idx_hbm.at[wid], idx_v)

        def gather(j, b):
            return pltpu.make_async_copy(
                table_hbm.at[idx_v.at[j]], rows_v.at[b], gsems[b]
            )

        def out_copy(j, b):
            return pltpu.make_async_copy(
                rows_v.at[b],
                out_hbm.at[pl.ds(base + j * CHUNK, CHUNK)],
                osems[b],
            )

        gather(0, 0).start()

        def body(j0, _):
            for b in (0, 1):
                j = j0 + b
                gather(j, b).wait()

                @pl.when(j >= 1)
                def _():
                    out_copy(j - 1, 1 - b).wait()

                out_copy(j, b).start()

                @pl.when(j + 1 < n_chunks)
                def _():
                    gather(j + 1, 1 - b).start()

            return ()

        lax.fori_loop(0, n_chunks // 2, lambda i, c: body(i * 2, c), ())
        out_copy(n_chunks - 1, (n_chunks - 1) % 2).wait()

    out = gather_kernel(idx, table)
    return out.reshape(batch, seq, d)


# speedup vs baseline: 4.1647x; 1.3340x over previous
import functools

import jax
import jax.numpy as jnp
from jax import lax
from jax.experimental import pallas as pl
from jax.experimental.pallas import tpu as pltpu
from jax.experimental.pallas import tpu_sc as plsc

D_MODEL = 128


@jax.jit
def kernel(x, table):
    batch, seq = x.shape
    d = table.shape[1]

    info = plsc.get_sparse_core_info()
    n_workers = info.num_cores * info.num_subcores
    rows_per_w = batch // n_workers
    assert rows_per_w * n_workers == batch

    xi = x.astype(jnp.int32)
    mesh = plsc.VectorSubcoreMesh(core_axis_name="c", subcore_axis_name="s")

    @functools.partial(
        pl.kernel,
        mesh=mesh,
        out_type=jax.ShapeDtypeStruct((batch, seq, d), jnp.float32),
        scratch_types=[
            pltpu.VMEM((rows_per_w, seq), jnp.int32),
            pltpu.VMEM((2, seq, D_MODEL), jnp.float32),
            pltpu.SemaphoreType.DMA,
            pltpu.SemaphoreType.DMA,
            pltpu.SemaphoreType.DMA,
            pltpu.SemaphoreType.DMA,
        ],
        compiler_params=pltpu.CompilerParams(use_tc_tiling_on_sc=True),
    )
    def gather_kernel(
        idx_hbm, table_hbm, out_hbm, idx_v, rows_v, g0, g1, o0, o1
    ):
        wid = lax.axis_index("s") * info.num_cores + lax.axis_index("c")
        base = wid * rows_per_w
        gsems = (g0, g1)
        osems = (o0, o1)
        pltpu.sync_copy(idx_hbm.at[pl.ds(base, rows_per_w)], idx_v)

        def gather(r, b):
            return pltpu.make_async_copy(
                table_hbm.at[idx_v.at[r]], rows_v.at[b], gsems[b]
            )

        def out_copy(r, b):
            return pltpu.make_async_copy(
                rows_v.at[b], out_hbm.at[base + r], osems[b]
            )

        gather(0, 0).start()

        def body(r0, _):
            for b in (0, 1):
                r = r0 + b
                gather(r, b).wait()

                @pl.when(r >= 1)
                def _():
                    out_copy(r - 1, 1 - b).wait()

                out_copy(r, b).start()

                @pl.when(r + 1 < rows_per_w)
                def _():
                    gather(r + 1, 1 - b).start()

            return ()

        lax.fori_loop(0, rows_per_w // 2, lambda i, c: body(i * 2, c), ())
        out_copy(rows_per_w - 1, (rows_per_w - 1) % 2).wait()

    return gather_kernel(xi, table)


# speedup vs baseline: 5.9554x; 1.4299x over previous
import functools

import jax
import jax.numpy as jnp
from jax import lax
from jax.experimental import pallas as pl
from jax.experimental.pallas import tpu as pltpu
from jax.experimental.pallas import tpu_sc as plsc

D_MODEL = 128


@jax.jit
def kernel(x, table):
    batch, seq = x.shape
    d = table.shape[1]

    info = plsc.get_sparse_core_info()
    n_workers = info.num_cores * info.num_subcores
    rows_per_w = batch // n_workers
    assert rows_per_w * n_workers == batch

    xi = x.astype(jnp.int32)
    mesh = plsc.VectorSubcoreMesh(core_axis_name="c", subcore_axis_name="s")

    ng = 4
    gr = 2
    n_groups = rows_per_w // gr
    assert n_groups % ng == 0

    @functools.partial(
        pl.kernel,
        mesh=mesh,
        out_type=jax.ShapeDtypeStruct((batch, seq, d), jnp.float32),
        scratch_types=[
            pltpu.VMEM((rows_per_w, seq), jnp.int32),
            pltpu.VMEM((ng, gr, seq, D_MODEL), jnp.float32),
            [pltpu.SemaphoreType.DMA] * ng,
            [pltpu.SemaphoreType.DMA] * ng,
        ],
        compiler_params=pltpu.CompilerParams(use_tc_tiling_on_sc=True),
    )
    def gather_kernel(idx_hbm, table_hbm, out_hbm, idx_v, rows_v, gsems, osems):
        wid = lax.axis_index("s") * info.num_cores + lax.axis_index("c")
        base = wid * rows_per_w
        pltpu.sync_copy(idx_hbm.at[pl.ds(base, rows_per_w)], idx_v)

        def gather_part(t, g, i):
            return pltpu.make_async_copy(
                table_hbm.at[idx_v.at[t * gr + i]],
                rows_v.at[g].at[i],
                gsems[g],
            )

        def gather_start(t, g):
            for i in range(gr):
                gather_part(t, g, i).start()

        def gather_wait(t, g):
            for i in range(gr):
                gather_part(t, g, i).wait()

        def out_copy(t, g):
            return pltpu.make_async_copy(
                rows_v.at[g],
                out_hbm.at[pl.ds(base + t * gr, gr)],
                osems[g],
            )

        gather_start(0, 0)

        def body(t0, _):
            for k in range(ng):
                t = t0 + k
                g = (k + 1) % ng

                @pl.when((t + 1 < n_groups) & (t + 1 >= ng))
                def _():
                    out_copy(t + 1 - ng, g).wait()

                @pl.when(t + 1 < n_groups)
                def _():
                    gather_start(t + 1, g)

                gather_wait(t, k)
                out_copy(t, k).start()

            return ()

        lax.fori_loop(0, n_groups // ng, lambda i, c: body(i * ng, c), ())
        for k in range(ng):
            out_copy(n_groups - ng + k, k).wait()

    return gather_kernel(xi, table)


# speedup vs baseline: 10.6167x; 1.7827x over previous
import functools

import jax
import jax.numpy as jnp
from jax import lax
from jax.experimental import pallas as pl
from jax.experimental.pallas import tpu as pltpu
from jax.experimental.pallas import tpu_sc as plsc

D_MODEL = 128


@jax.jit
def kernel(x, table):
    batch, seq = x.shape
    d = table.shape[1]

    info = plsc.get_sparse_core_info()
    n_workers = info.num_cores * info.num_subcores
    bw = batch // n_workers
    assert bw * n_workers == batch

    xt = x.astype(jnp.int32).T
    mesh = plsc.VectorSubcoreMesh(core_axis_name="c", subcore_axis_name="s")

    ng = 5
    assert seq % ng == 0

    @functools.partial(
        pl.kernel,
        mesh=mesh,
        out_type=jax.ShapeDtypeStruct((seq, batch, d), jnp.float32),
        scratch_types=[
            pltpu.VMEM((seq, bw), jnp.int32),
            pltpu.VMEM((ng, bw, D_MODEL), jnp.float32),
            [pltpu.SemaphoreType.DMA] * ng,
            [pltpu.SemaphoreType.DMA] * ng,
        ],
        compiler_params=pltpu.CompilerParams(use_tc_tiling_on_sc=True),
    )
    def gather_kernel(idx_hbm, table_hbm, out_hbm, idx_v, rows_v, gsems, osems):
        wid = lax.axis_index("s") * info.num_cores + lax.axis_index("c")
        base = wid * bw
        pltpu.sync_copy(idx_hbm.at[:, pl.ds(base, bw)], idx_v)

        def gather(t, g):
            return pltpu.make_async_copy(
                table_hbm.at[idx_v.at[t]], rows_v.at[g], gsems[g]
            )

        def out_copy(t, g):
            return pltpu.make_async_copy(
                rows_v.at[g], out_hbm.at[t, pl.ds(base, bw)], osems[g]
            )

        gather(0, 0).start()

        def body(t0, _):
            for k in range(ng):
                t = t0 + k
                g = (k + 1) % ng

                @pl.when((t + 1 < seq) & (t + 1 >= ng))
                def _():
                    out_copy(t + 1 - ng, g).wait()

                @pl.when(t + 1 < seq)
                def _():
                    gather(t + 1, g).start()

                gather(t, k).wait()
                out_copy(t, k).start()

            return ()

        lax.fori_loop(0, seq // ng, lambda i, c: body(i * ng, c), ())
        for k in range(ng):
            out_copy(seq - ng + k, k).wait()

    out = gather_kernel(xt, table)
    return out.transpose(1, 0, 2)


# speedup vs baseline: 10.6480x; 1.0029x over previous
import functools

import jax
import jax.numpy as jnp
from jax import lax
from jax.experimental import pallas as pl
from jax.experimental.pallas import tpu as pltpu
from jax.experimental.pallas import tpu_sc as plsc

D_MODEL = 128


@jax.jit
def kernel(x, table):
    batch, seq = x.shape
    d = table.shape[1]

    info = plsc.get_sparse_core_info()
    n_workers = info.num_cores * info.num_subcores
    bw = batch // n_workers
    assert bw * n_workers == batch

    xt = x.astype(jnp.int32).T
    mesh = plsc.VectorSubcoreMesh(core_axis_name="c", subcore_axis_name="s")

    ng = 5
    assert seq % ng == 0

    @functools.partial(
        pl.kernel,
        mesh=mesh,
        out_type=jax.ShapeDtypeStruct((seq, batch, d), jnp.float32),
        scratch_types=[
            pltpu.VMEM((seq, bw), jnp.int32),
            pltpu.VMEM((ng, bw, D_MODEL), jnp.float32),
            [pltpu.SemaphoreType.DMA] * ng,
            [pltpu.SemaphoreType.DMA] * ng,
        ],
        compiler_params=pltpu.CompilerParams(use_tc_tiling_on_sc=True),
    )
    def gather_kernel(idx_hbm, table_hbm, out_hbm, idx_v, rows_v, gsems, osems):
        wid = lax.axis_index("s") * info.num_cores + lax.axis_index("c")
        base = wid * bw
        pltpu.sync_copy(idx_hbm.at[:, pl.ds(base, bw)], idx_v)

        def gather(t, g):
            return pltpu.make_async_copy(
                table_hbm.at[idx_v.at[t]], rows_v.at[g], gsems[g]
            )

        def out_copy(t, g):
            return pltpu.make_async_copy(
                rows_v.at[g], out_hbm.at[t, pl.ds(base, bw)], osems[g]
            )

        gather(0, 0).start()
        gather(1, 1).start()

        def body(t0, _):
            for k in range(ng):
                t = t0 + k
                g = (k + 2) % ng

                @pl.when((t + 2 < seq) & (t + 2 >= ng))
                def _():
                    out_copy(t + 2 - ng, g).wait()

                @pl.when(t + 2 < seq)
                def _():
                    gather(t + 2, g).start()

                gather(t, k).wait()
                out_copy(t, k).start()

            return ()

        lax.fori_loop(0, seq // ng, lambda i, c: body(i * ng, c), ())
        for k in range(ng):
            out_copy(seq - ng + k, k).wait()

    out = gather_kernel(xt, table)
    return out.transpose(1, 0, 2)
